# baseline (speedup 1.0000x reference)
import functools

import jax
import jax.numpy as jnp
from jax import lax
from jax.experimental import pallas as pl
from jax.experimental.pallas import tpu as pltpu
from jax.experimental.pallas import tpu_sc as plsc

_NUM_CORES = 2
_NUM_SUBCORES = 16
_NW = _NUM_CORES * _NUM_SUBCORES
_LANES = 16


def _splat(v):
  return jnp.full((_LANES,), v, jnp.int32)


def _sc_build_pos(dims, row_w, col_w, hei_w):
  B, Z, C, X, Y = dims
  K = X * Y
  CH = col_w.shape[1]
  CH2 = 2 * CH
  RPW = C // _NW
  CHUNK = RPW * K

  mesh = plsc.VectorSubcoreMesh(
      core_axis_name="c", subcore_axis_name="s",
      num_cores=_NUM_CORES, num_subcores=_NUM_SUBCORES)

  @functools.partial(
      pl.kernel,
      out_type=jax.ShapeDtypeStruct((Z * C * K,), jnp.float32),
      mesh=mesh,
      scratch_types=[
          pltpu.VMEM((row_w.size,), jnp.float32),
          pltpu.VMEM((col_w.size,), jnp.float32),
          pltpu.VMEM((hei_w.size,), jnp.float32),
          pltpu.VMEM((Z * CHUNK,), jnp.float32),
          pltpu.SemaphoreType.DMA,
      ],
      compiler_params=pltpu.CompilerParams(needs_layout_passes=False),
  )
  def kern(row_hbm, col_hbm, hei_hbm, pos_hbm, roww_v, colw_v, heiw_v,
           slabs_v, sem):
    cid = lax.axis_index("c")
    sid = lax.axis_index("s")
    wid = sid * _NUM_CORES + cid
    c0 = wid * RPW
    zdep = c0 + RPW > CH2
    nslab = jnp.where(zdep, Z, 1)

    pltpu.sync_copy(row_hbm, roww_v)
    pltpu.sync_copy(col_hbm, colw_v)
    pltpu.sync_copy(hei_hbm, heiw_v)

    idx16 = lax.iota(jnp.int32, 16)

    def build_row(z, r):
      c = c0 + r
      off = (z * RPW + r) * K

      def col_case():
        ga = plsc.load_gather(colw_v, [idx16 * CH + c])
        gb = plsc.load_gather(colw_v, [(idx16 + 16) * CH + c])

        def st(m, _):
          slabs_v[pl.ds(off + 32 * m, 16)] = ga
          slabs_v[pl.ds(off + 32 * m + 16, 16)] = gb
          return 0
        lax.fori_loop(0, X, st, 0)

      def row_case():
        def st(x, _):
          s = plsc.load_gather(roww_v, [_splat(x * CH + c - CH)])
          slabs_v[pl.ds(off + 32 * x, 16)] = s
          slabs_v[pl.ds(off + 32 * x + 16, 16)] = s
          return 0
        lax.fori_loop(0, X, st, 0)

      def hei_case():
        s = plsc.load_gather(heiw_v, [_splat(z * CH + c - CH2)])

        def st(m, _):
          slabs_v[pl.ds(off + 16 * m, 16)] = s
          return 0
        lax.fori_loop(0, K // 16, st, 0)

      lax.cond(c < CH, col_case,
               lambda: lax.cond(c < CH2, row_case, hei_case))

    def build_slab(z, _):
      def row_body(r, _):
        build_row(z, r)
        return 0
      lax.fori_loop(0, RPW, row_body, 0)
      return 0

    lax.fori_loop(0, nslab, build_slab, 0)

    def issue(z, _):
      zs = jnp.where(zdep, z, 0)
      pltpu.async_copy(
          slabs_v.at[pl.ds(zs * CHUNK, CHUNK)],
          pos_hbm.at[pl.ds(z * C * K + c0 * K, CHUNK)], sem)
      return 0

    lax.fori_loop(0, Z, issue, 0)

    def drain(z, _):
      pltpu.make_async_copy(
          slabs_v.at[pl.ds(0, CHUNK)],
          pos_hbm.at[pl.ds(c0 * K, CHUNK)], sem).wait()
      return 0

    lax.fori_loop(0, Z, drain, 0)

  return kern(row_w.reshape(-1), col_w.reshape(-1), hei_w.reshape(-1))


def _tc_broadcast(dims, pos):
  B, Z, C, X, Y = dims
  CK = C * X * Y
  WINDOW = 24

  def body(pos_hbm, out_hbm, pos_v, sem_in, sem_out):
    pltpu.make_async_copy(pos_hbm, pos_v, sem_in).start()
    pltpu.make_async_copy(pos_hbm, pos_v, sem_in).wait()

    def wait_one():
      pltpu.make_async_copy(
          pos_v.at[pl.ds(0, 1)], out_hbm.at[pl.ds(0, 1)], sem_out).wait()

    def issue(t, _):
      b = t // Z
      z = t - b * Z
      pltpu.make_async_copy(
          pos_v.at[pl.ds(z, 1)], out_hbm.at[pl.ds(b * Z + z, 1)],
          sem_out).start()

      @pl.when(t >= WINDOW)
      def _():
        wait_one()
      return 0

    lax.fori_loop(0, B * Z, issue, 0, unroll=4)

    def drain(t, _):
      wait_one()
      return 0

    lax.fori_loop(0, min(WINDOW, B * Z), drain, 0)

  out = pl.pallas_call(
      body,
      out_shape=jax.ShapeDtypeStruct((B * Z, CK), jnp.float32),
      in_specs=[pl.BlockSpec(memory_space=pl.ANY)],
      out_specs=pl.BlockSpec(memory_space=pl.ANY),
      scratch_shapes=[
          pltpu.VMEM((Z, CK), jnp.float32),
          pltpu.SemaphoreType.DMA,
          pltpu.SemaphoreType.DMA,
      ],
  )(pos.reshape(Z, CK))
  return out


@functools.partial(jax.jit, static_argnums=(0,))
def _pos_embed(dims, row_w, col_w, hei_w):
  B, Z, C, X, Y = dims
  pos = _sc_build_pos(dims, row_w, col_w, hei_w)
  out = _tc_broadcast(dims, pos)
  return out.reshape(B, Z, C, X, Y)


def kernel(tensor, row_w, col_w, hei_w):
  B, Z, C, X, Y = tensor.shape
  assert C % _NW == 0 and X == 32 and Y == 32
  return _pos_embed((B, Z, C, X, Y), row_w, col_w, hei_w)


# speedup vs baseline: 9.0841x; 9.0841x over previous
import functools

import jax
import jax.numpy as jnp
from jax import lax
from jax.experimental import pallas as pl
from jax.experimental.pallas import tpu as pltpu
from jax.experimental.pallas import tpu_sc as plsc

_NUM_CORES = 2
_NUM_SUBCORES = 16
_NW = _NUM_CORES * _NUM_SUBCORES


def _chunk_starts(lo, hi):
  starts = []
  b = lo // 128
  while b * 128 < hi:
    s0, s1 = max(lo, b * 128), min(hi, (b + 1) * 128)
    seg = list(range(s0, s1 - 16, 16))
    seg.append(s1 - 16)
    starts += seg
    b += 1
  return starts


def _sc_build_pos(dims, row_w, col_w, hei_w):
  B, Z, C, X, Y = dims
  CH = col_w.shape[1]
  CH2 = 2 * CH
  CHZ = C - CH2

  mesh = plsc.VectorSubcoreMesh(
      core_axis_name="c", subcore_axis_name="s",
      num_cores=_NUM_CORES, num_subcores=_NUM_SUBCORES)

  @functools.partial(
      pl.kernel,
      out_type=jax.ShapeDtypeStruct((Z, X, Y, C), jnp.float32),
      mesh=mesh,
      scratch_types=[
          pltpu.VMEM((row_w.size,), jnp.float32),
          pltpu.VMEM((col_w.size,), jnp.float32),
          pltpu.VMEM((hei_w.size,), jnp.float32),
          pltpu.VMEM((Z * Y, C), jnp.float32),
          pltpu.SemaphoreType.DMA,
      ],
      compiler_params=pltpu.CompilerParams(needs_layout_passes=False),
  )
  def kern(row_hbm, col_hbm, hei_hbm, pos_hbm, roww_v, colw_v, heiw_v,
           slabs_v, sem):
    cid = lax.axis_index("c")
    sid = lax.axis_index("s")
    wid = sid * _NUM_CORES + cid

    pltpu.sync_copy(row_hbm, roww_v)
    pltpu.sync_copy(col_hbm, colw_v)
    pltpu.sync_copy(hei_hbm, heiw_v)

    col_s = _chunk_starts(0, CH)
    row_s = _chunk_starts(CH, CH2)
    hei_s = _chunk_starts(CH2, C)

    rw = [roww_v[pl.ds(wid * CH + (s - CH), 16)] for s in row_s]

    def per_z(z, _):
      hz = [heiw_v[pl.ds(z * CH + (s - CH2), 16)] for s in hei_s]

      def per_y(y, _):
        r = z * Y + y
        for s in col_s:
          slabs_v[r, pl.ds(s, 16)] = colw_v[pl.ds(y * CH + s, 16)]
        for v, s in zip(rw, row_s):
          slabs_v[r, pl.ds(s, 16)] = v
        for v, s in zip(hz, hei_s):
          slabs_v[r, pl.ds(s, 16)] = v
        return 0

      lax.fori_loop(0, Y, per_y, 0)
      return 0

    lax.fori_loop(0, Z, per_z, 0)

    def issue(z, _):
      pltpu.async_copy(
          slabs_v.at[pl.ds(z * Y, Y)], pos_hbm.at[z, wid], sem)
      return 0

    lax.fori_loop(0, Z, issue, 0)

    def drain(z, _):
      pltpu.make_async_copy(
          slabs_v.at[pl.ds(0, Y)], pos_hbm.at[0, wid], sem).wait()
      return 0

    lax.fori_loop(0, Z, drain, 0)

  return kern(row_w.reshape(-1), col_w.reshape(-1), hei_w.reshape(-1))


def _tc_broadcast(dims, pos):
  B, Z, C, X, Y = dims
  WINDOW = 24

  def body(pos_hbm, out_hbm, pos_v, sem_in, sem_out):
    cp = pltpu.make_async_copy(pos_hbm, pos_v, sem_in)
    cp.start()
    cp.wait()

    def wait_one():
      pltpu.make_async_copy(
          pos_v.at[0], out_hbm.at[0, 0], sem_out).wait()

    def issue(t, _):
      b = t // Z
      z = t - b * Z
      pltpu.make_async_copy(
          pos_v.at[z], out_hbm.at[b, z], sem_out).start()

      @pl.when(t >= WINDOW)
      def _():
        wait_one()
      return 0

    lax.fori_loop(0, B * Z, issue, 0, unroll=4)

    def drain(t, _):
      wait_one()
      return 0

    lax.fori_loop(0, min(WINDOW, B * Z), drain, 0)

  return pl.pallas_call(
      body,
      out_shape=jax.ShapeDtypeStruct((B, Z, X, Y, C), jnp.float32),
      in_specs=[pl.BlockSpec(memory_space=pl.ANY)],
      out_specs=pl.BlockSpec(memory_space=pl.ANY),
      scratch_shapes=[
          pltpu.VMEM((Z, X, Y, C), jnp.float32),
          pltpu.SemaphoreType.DMA,
          pltpu.SemaphoreType.DMA,
      ],
  )(pos)


@functools.partial(jax.jit, static_argnums=(0,))
def _pos_embed(dims, row_w, col_w, hei_w):
  pos = _sc_build_pos(dims, row_w, col_w, hei_w)
  out = _tc_broadcast(dims, pos)
  return jnp.transpose(out, (0, 1, 4, 2, 3))


def kernel(tensor, row_w, col_w, hei_w):
  B, Z, C, X, Y = tensor.shape
  assert X == _NW and Y == X and C > 2 * row_w.shape[1]
  return _pos_embed((B, Z, C, X, Y), row_w, col_w, hei_w)
